# baseline (speedup 1.0000x reference)
import jax
import jax.numpy as jnp
from jax import lax
from jax.experimental import pallas as pl
from jax.experimental.pallas import tpu as pltpu
from jax.experimental.pallas import tpu_sc as plsc

N = 10000
E = 320000
D = 128
HD = D // 2
ATT = 256
NC = 2
NS = 16
K = 80
CH = E // (NS * K)
WPT = 624
ZB = 16


def _scores_body(x_ref, wc_ref, bc_ref, wr_ref, br_ref, out_ref):
    c = jnp.dot(x_ref[...], wc_ref[...], preferred_element_type=jnp.float32)
    c = c + bc_ref[...][None, :]
    s = jnp.dot(c, wr_ref[...], preferred_element_type=jnp.float32)
    out_ref[...] = s + br_ref[...][None, :]


def _scores(x, W_coef, b_coef, wr2, br2):
    return pl.pallas_call(
        _scores_body,
        out_shape=jax.ShapeDtypeStruct((N, 2), jnp.float32),
    )(x, W_coef, b_coef, wr2, br2)


EPT = E // NS


def _sc_body(s_hbm, ei_hbm, xh_hbm, u_hbm, dn_hbm,
             s12_v, srcf, dstf, gbuf0, gbuf1, dbuf0, dbuf1, exb0, exb1,
             rb0, rb1, rows0, rows1, dn_v, zbuf, ush,
             gsem0, gsem1, ssem0, ssem1):
    c = lax.axis_index("c")
    s = lax.axis_index("s")
    ci = c.astype(jnp.int32)

    zeros16 = jnp.zeros((16,), jnp.float32)

    @pl.loop(0, ZB)
    def _zero_zbuf(i):
        for cc in range(HD // 16):
            zbuf[i, pl.ds(cc * 16, 16)] = zeros16

    @pl.loop(0, (N // 16))
    def _zero_dn(i):
        dn_v[pl.ds(i * 16, 16)] = zeros16

    for t in range(WPT // ZB):
        pltpu.sync_copy(zbuf, ush.at[pl.ds(s * WPT + t * ZB, ZB)])

    @pl.when(s == 0)
    def _zero_tail():
        pltpu.sync_copy(zbuf.at[pl.ds(0, N - NS * WPT)],
                        ush.at[pl.ds(NS * WPT, N - NS * WPT)])

    plsc.subcore_barrier()

    pltpu.sync_copy(s_hbm, s12_v)
    pltpu.sync_copy(ei_hbm.at[0, pl.ds(s * EPT, EPT)], srcf)
    pltpu.sync_copy(ei_hbm.at[1, pl.ds(s * EPT, EPT)], dstf)

    sets = ((gbuf0, dbuf0, exb0, rb0, rows0, gsem0, ssem0),
            (gbuf1, dbuf1, exb1, rb1, rows1, gsem1, ssem1))

    def prep(jj, gb, db, eb):
        for i in range(K // 16):
            sl = pl.ds(i * 16, 16)
            esl = pl.ds(jj * K + i * 16, 16)
            dv = dstf[esl]
            sv = srcf[esl]
            sv2 = sv + sv
            gb[sl] = sv2 + ci
            db[sl] = dv
            ex = jnp.exp(plsc.load_gather(s12_v, [dv + dv])
                         + plsc.load_gather(s12_v, [sv2 + 1]))
            eb[sl] = ex

            @pl.when(c == 0)
            def _dn():
                plsc.addupdate_scatter(dn_v, [dv], ex)

    def scale(rw, rb, eb):
        for i in range(K // 16):
            ev = eb[pl.ds(i * 16, 16)]
            for rr in range(16):
                e = ev[rr]
                r = i * 16 + rr
                for cc in range(HD // 32):
                    v = rb[r, pl.ds(cc * 32, 32)]
                    a, b = plsc.unpack(v, format=plsc.PackFormat.INTERLEAVED,
                                       preferred_element_type=jnp.float32)
                    rw[r, pl.ds(cc * 32, 16)] = a * e
                    rw[r, pl.ds(cc * 32 + 16, 16)] = b * e

    prep(0, gbuf0, dbuf0, exb0)
    pltpu.make_async_copy(xh_hbm.at[gbuf0], rb0, gsem0).start()

    @pl.loop(0, CH, step=2)
    def _pair(j0):
        for p in range(2):
            j = j0 + p
            gb, db, eb, rb, rw, gs, ss = sets[p]
            gbn, dbn, ebn, rbn, rwn, gsn, ssn = sets[1 - p]

            @pl.when(j >= 1)
            def _wait_prev_scatter():
                pltpu.make_async_copy(rwn, ush.at[dbn], ssn).wait()

            @pl.when(j + 1 < CH)
            def _prep_next():
                prep(j + 1, gbn, dbn, ebn)
                pltpu.make_async_copy(xh_hbm.at[gbn], rbn, gsn).start()

            pltpu.make_async_copy(xh_hbm.at[gb], rb, gs).wait()
            scale(rw, rb, eb)
            pltpu.make_async_copy(rw, ush.at[db], ss).start(add=True)

    pltpu.make_async_copy(rows1, ush.at[dbuf1], ssem1).wait()

    plsc.subcore_barrier()

    pltpu.sync_copy(ush.at[pl.ds(s * WPT, WPT)], u_hbm.at[c, pl.ds(s * WPT, WPT)])

    @pl.when(s == 0)
    def _write_tail():
        pltpu.sync_copy(ush.at[pl.ds(NS * WPT, N - NS * WPT)],
                        u_hbm.at[c, pl.ds(NS * WPT, N - NS * WPT)])

    @pl.when(c == 0)
    def _write_dn():
        pltpu.sync_copy(dn_v, dn_hbm.at[s, 0])


def _sc_aggregate(sc, ei, xh):
    kfn = pl.kernel(
        _sc_body,
        out_type=(jax.ShapeDtypeStruct((NC, N, HD), jnp.float32),
                  jax.ShapeDtypeStruct((NS, 1, N), jnp.float32)),
        mesh=plsc.VectorSubcoreMesh(core_axis_name="c", subcore_axis_name="s"),
        compiler_params=pltpu.CompilerParams(needs_layout_passes=False,
                                             use_tc_tiling_on_sc=False),
        scratch_types=(
            pltpu.VMEM((2 * N,), jnp.float32),
            pltpu.VMEM((EPT,), jnp.int32),
            pltpu.VMEM((EPT,), jnp.int32),
            pltpu.VMEM((K,), jnp.int32),
            pltpu.VMEM((K,), jnp.int32),
            pltpu.VMEM((K,), jnp.int32),
            pltpu.VMEM((K,), jnp.int32),
            pltpu.VMEM((K,), jnp.float32),
            pltpu.VMEM((K,), jnp.float32),
            pltpu.VMEM((K, HD), jnp.bfloat16),
            pltpu.VMEM((K, HD), jnp.bfloat16),
            pltpu.VMEM((K, HD), jnp.float32),
            pltpu.VMEM((K, HD), jnp.float32),
            pltpu.VMEM((N,), jnp.float32),
            pltpu.VMEM((ZB, HD), jnp.float32),
            pltpu.VMEM_SHARED((N, HD), jnp.float32),
            pltpu.SemaphoreType.DMA,
            pltpu.SemaphoreType.DMA,
            pltpu.SemaphoreType.DMA,
            pltpu.SemaphoreType.DMA,
        ),
    )
    return kfn(sc, ei, xh)


def _final_body(x_ref, u_ref, dn_ref, wa_ref, ba_ref, out_ref):
    den = jnp.sum(dn_ref[0], axis=0) + 1e-9
    neigh = jnp.concatenate([u_ref[0], u_ref[1]], axis=-1) / den[:, None]
    h = jnp.dot(x_ref[...], wa_ref[:D], preferred_element_type=jnp.float32)
    h = h + jnp.dot(neigh, wa_ref[D:], preferred_element_type=jnp.float32)
    h = h + ba_ref[...][None, :]
    nrm = jnp.sqrt(jnp.sum(h * h, axis=-1, keepdims=True))
    out_ref[...] = h / jnp.maximum(nrm, 1e-12)


def _final(x, u, dn, W_att, b_att):
    blk = 2000
    grid = N // blk
    return pl.pallas_call(
        _final_body,
        grid=(grid,),
        in_specs=[
            pl.BlockSpec((blk, D), lambda i: (i, 0)),
            pl.BlockSpec((NC, blk, HD), lambda i: (0, i, 0)),
            pl.BlockSpec((1, NS, blk), lambda i: (i, 0, 0)),
            pl.BlockSpec((2 * D, ATT), lambda i: (0, 0)),
            pl.BlockSpec((ATT,), lambda i: (0,)),
        ],
        out_specs=pl.BlockSpec((blk, ATT), lambda i: (i, 0)),
        out_shape=jax.ShapeDtypeStruct((N, ATT), jnp.float32),
    )(x, u, dn.reshape(NS, N // blk, blk).transpose(1, 0, 2), W_att, b_att)


def kernel(x, edge_index, W_coef, b_coef, W_att, b_att, W_red, b_red):
    wr2 = jnp.concatenate([W_red[:D], W_red[D:]], axis=1)
    br2 = jnp.concatenate([b_red, jnp.zeros((1,), jnp.float32)])
    sc = _scores(x, W_coef, b_coef, wr2, br2)
    xh = (x.astype(jnp.bfloat16)
           .reshape(N, 2, 2, 2, 16)
           .transpose(0, 1, 2, 4, 3)
           .reshape(2 * N, HD))
    u, dn = _sc_aggregate(sc.reshape(2 * N), edge_index, xh)
    return _final(x, u, dn.reshape(NS, N), W_att, b_att)


# speedup vs baseline: 2.4097x; 2.4097x over previous
import jax
import jax.numpy as jnp
from jax import lax
from jax.experimental import pallas as pl
from jax.experimental.pallas import tpu as pltpu
from jax.experimental.pallas import tpu_sc as plsc

N = 10000
E = 320000
D = 128
HD = D // 2
ATT = 256
NC = 2
NS = 16
K = 80
CH = E // (NS * K)
WPT = 624
ZB = 16


def _scores_body(x_ref, wc_ref, bc_ref, wr_ref, br_ref, out_ref):
    c = jnp.dot(x_ref[...], wc_ref[...], preferred_element_type=jnp.float32)
    c = c + bc_ref[...][None, :]
    s = jnp.dot(c, wr_ref[...], preferred_element_type=jnp.float32)
    out_ref[...] = s + br_ref[...][None, :]


def _scores(x, W_coef, b_coef, wr2, br2):
    return pl.pallas_call(
        _scores_body,
        out_shape=jax.ShapeDtypeStruct((N, 2), jnp.float32),
    )(x, W_coef, b_coef, wr2, br2)


EPT = E // NS


def _sc_body(s_hbm, ei_hbm, xh_hbm, u_hbm, dn_hbm,
             s12_v, srcf, dstf, gbuf0, gbuf1, dbuf0, dbuf1, exb0, exb1,
             rows0, rows1, dn_v, zbuf, ush,
             gsem0, gsem1, ssem0, ssem1):
    c = lax.axis_index("c")
    s = lax.axis_index("s")
    ci = c.astype(jnp.int32)

    zeros16 = jnp.zeros((16,), jnp.float32)

    @pl.loop(0, ZB)
    def _zero_zbuf(i):
        for cc in range(HD // 16):
            zbuf[i, pl.ds(cc * 16, 16)] = zeros16

    @pl.loop(0, (N // 16))
    def _zero_dn(i):
        dn_v[pl.ds(i * 16, 16)] = zeros16

    for t in range(WPT // ZB):
        pltpu.sync_copy(zbuf, ush.at[pl.ds(s * WPT + t * ZB, ZB)])

    @pl.when(s == 0)
    def _zero_tail():
        pltpu.sync_copy(zbuf.at[pl.ds(0, N - NS * WPT)],
                        ush.at[pl.ds(NS * WPT, N - NS * WPT)])

    plsc.subcore_barrier()

    pltpu.sync_copy(s_hbm, s12_v)
    pltpu.sync_copy(ei_hbm.at[0, pl.ds(s * EPT, EPT)], srcf)
    pltpu.sync_copy(ei_hbm.at[1, pl.ds(s * EPT, EPT)], dstf)

    sets = ((gbuf0, dbuf0, exb0, rows0, gsem0, ssem0),
            (gbuf1, dbuf1, exb1, rows1, gsem1, ssem1))

    def prep(jj, gb, eb):
        for i in range(K // 16):
            sl = pl.ds(i * 16, 16)
            esl = pl.ds(jj * K + i * 16, 16)
            dv = dstf[esl]
            sv = srcf[esl]
            sv2 = sv + sv
            gb[sl] = sv2 + ci
            ex = jnp.exp(plsc.load_gather(s12_v, [dv + dv])
                         + plsc.load_gather(s12_v, [sv2 + 1]))
            eb[sl] = ex

            @pl.when(c == 0)
            def _dn():
                plsc.addupdate_scatter(dn_v, [dv], ex)

    def write_db(jj, db):
        for i in range(K // 16):
            db[pl.ds(i * 16, 16)] = dstf[pl.ds(jj * K + i * 16, 16)]

    def scale(rw, eb):
        for i in range(K // 16):
            ev = eb[pl.ds(i * 16, 16)]
            for rr in range(16):
                e = ev[rr]
                r = i * 16 + rr
                for cc in range(HD // 16):
                    sl = pl.ds(cc * 16, 16)
                    rw[r, sl] = rw[r, sl] * e

    prep(0, gbuf0, exb0)
    prep(1, gbuf1, exb1)
    write_db(0, dbuf0)
    pltpu.make_async_copy(xh_hbm.at[gbuf0], rows0, gsem0).start()

    @pl.loop(0, CH, step=2)
    def _pair(j0):
        for p in range(2):
            j = j0 + p
            gb, db, eb, rw, gs, ss = sets[p]
            gbn, dbn, ebn, rwn, gsn, ssn = sets[1 - p]

            @pl.when(j >= 1)
            def _wait_prev_scatter():
                pltpu.make_async_copy(rwn, ush.at[dbn], ssn).wait()

            @pl.when(j + 1 < CH)
            def _fire_next_gather():
                write_db(j + 1, dbn)
                pltpu.make_async_copy(xh_hbm.at[gbn], rwn, gsn).start()

            pltpu.make_async_copy(xh_hbm.at[gb], rw, gs).wait()
            scale(rw, eb)
            pltpu.make_async_copy(rw, ush.at[db], ss).start(add=True)

            @pl.when(j + 2 < CH)
            def _prep_ahead():
                prep(j + 2, gb, eb)

    pltpu.make_async_copy(rows1, ush.at[dbuf1], ssem1).wait()

    plsc.subcore_barrier()

    pltpu.sync_copy(ush.at[pl.ds(s * WPT, WPT)], u_hbm.at[c, pl.ds(s * WPT, WPT)])

    @pl.when(s == 0)
    def _write_tail():
        pltpu.sync_copy(ush.at[pl.ds(NS * WPT, N - NS * WPT)],
                        u_hbm.at[c, pl.ds(NS * WPT, N - NS * WPT)])

    @pl.when(c == 0)
    def _write_dn():
        pltpu.sync_copy(dn_v, dn_hbm.at[s, 0])


def _sc_aggregate(sc, ei, xh):
    kfn = pl.kernel(
        _sc_body,
        out_type=(jax.ShapeDtypeStruct((NC, N, HD), jnp.float32),
                  jax.ShapeDtypeStruct((NS, 1, N), jnp.float32)),
        mesh=plsc.VectorSubcoreMesh(core_axis_name="c", subcore_axis_name="s"),
        compiler_params=pltpu.CompilerParams(needs_layout_passes=False,
                                             use_tc_tiling_on_sc=False),
        scratch_types=(
            pltpu.VMEM((2 * N,), jnp.float32),
            pltpu.VMEM((EPT,), jnp.int32),
            pltpu.VMEM((EPT,), jnp.int32),
            pltpu.VMEM((K,), jnp.int32),
            pltpu.VMEM((K,), jnp.int32),
            pltpu.VMEM((K,), jnp.int32),
            pltpu.VMEM((K,), jnp.int32),
            pltpu.VMEM((K,), jnp.float32),
            pltpu.VMEM((K,), jnp.float32),
            pltpu.VMEM((K, HD), jnp.float32),
            pltpu.VMEM((K, HD), jnp.float32),
            pltpu.VMEM((N,), jnp.float32),
            pltpu.VMEM((ZB, HD), jnp.float32),
            pltpu.VMEM_SHARED((N, HD), jnp.float32),
            pltpu.SemaphoreType.DMA,
            pltpu.SemaphoreType.DMA,
            pltpu.SemaphoreType.DMA,
            pltpu.SemaphoreType.DMA,
        ),
    )
    return kfn(sc, ei, xh)


def _final_body(x_ref, u_ref, dn_ref, wa_ref, ba_ref, out_ref):
    den = jnp.sum(dn_ref[0], axis=0) + 1e-9
    neigh = jnp.concatenate([u_ref[0], u_ref[1]], axis=-1) / den[:, None]
    h = jnp.dot(x_ref[...], wa_ref[:D], preferred_element_type=jnp.float32)
    h = h + jnp.dot(neigh, wa_ref[D:], preferred_element_type=jnp.float32)
    h = h + ba_ref[...][None, :]
    nrm = jnp.sqrt(jnp.sum(h * h, axis=-1, keepdims=True))
    out_ref[...] = h / jnp.maximum(nrm, 1e-12)


def _final(x, u, dn, W_att, b_att):
    blk = 2000
    grid = N // blk
    return pl.pallas_call(
        _final_body,
        grid=(grid,),
        in_specs=[
            pl.BlockSpec((blk, D), lambda i: (i, 0)),
            pl.BlockSpec((NC, blk, HD), lambda i: (0, i, 0)),
            pl.BlockSpec((1, NS, blk), lambda i: (i, 0, 0)),
            pl.BlockSpec((2 * D, ATT), lambda i: (0, 0)),
            pl.BlockSpec((ATT,), lambda i: (0,)),
        ],
        out_specs=pl.BlockSpec((blk, ATT), lambda i: (i, 0)),
        out_shape=jax.ShapeDtypeStruct((N, ATT), jnp.float32),
    )(x, u, dn.reshape(NS, N // blk, blk).transpose(1, 0, 2), W_att, b_att)


def kernel(x, edge_index, W_coef, b_coef, W_att, b_att, W_red, b_red):
    wr2 = jnp.concatenate([W_red[:D], W_red[D:]], axis=1)
    br2 = jnp.concatenate([b_red, jnp.zeros((1,), jnp.float32)])
    sc = _scores(x, W_coef, b_coef, wr2, br2)
    xh = x.reshape(2 * N, HD)
    u, dn = _sc_aggregate(sc.reshape(2 * N), edge_index, xh)
    return _final(x, u, dn.reshape(NS, N), W_att, b_att)
